# baseline (speedup 1.0000x reference)
import functools

import jax
import jax.numpy as jnp
from jax import lax
from jax.experimental import pallas as pl
from jax.experimental.pallas import tpu as pltpu
from jax.experimental.pallas import tpu_sc as plsc

N_NODES = 10000
N_PAD = 10240
N_EDGES = 320000
IN_CH = 128
NC = 2
NS = 16
NW = NC * NS
WIN = 128
NWIN = 80
EPW = NWIN * WIN
RPT = N_PAD // NS

_MESH = plsc.VectorSubcoreMesh(core_axis_name="c", subcore_axis_name="s")


def _dot_b3(a, b):
    ah = a.astype(jnp.bfloat16)
    al = (a - ah.astype(jnp.float32)).astype(jnp.bfloat16)
    bh = b.astype(jnp.bfloat16)
    bl = (b - bh.astype(jnp.float32)).astype(jnp.bfloat16)
    d = lambda u, v: jnp.dot(u, v, preferred_element_type=jnp.float32)
    return d(ah, bh) + d(ah, bl) + d(al, bh)


def _sc_degree(e4):
    @functools.partial(
        pl.kernel,
        out_type=jax.ShapeDtypeStruct((NC, N_PAD), jnp.float32),
        mesh=_MESH,
        scratch_types=[
            pltpu.VMEM_SHARED((N_PAD,), jnp.float32),
            pltpu.VMEM((NWIN, WIN), jnp.int32),
            pltpu.VMEM((1, WIN), jnp.float32),
            pltpu.VMEM((RPT,), jnp.float32),
            pltpu.SemaphoreType.DMA,
        ],
    )
    def k(e_hbm, out_hbm, acc, idx_v, ones_v, z_v, sem):
        c = lax.axis_index("c")
        s = lax.axis_index("s")
        wid = c * NS + s
        pltpu.async_copy(e_hbm.at[1, wid], idx_v, sem)

        @pl.loop(0, WIN, step=16)
        def _(i):
            ones_v[0, pl.ds(i, 16)] = jnp.ones((16,), jnp.float32)

        @pl.loop(0, RPT, step=16)
        def _(i):
            z_v[pl.ds(i, 16)] = jnp.zeros((16,), jnp.float32)

        pltpu.sync_copy(z_v, acc.at[pl.ds(s * RPT, RPT)])
        plsc.subcore_barrier()
        pltpu.make_async_copy(e_hbm.at[1, wid], idx_v, sem).wait()

        @pl.loop(0, NWIN, step=8)
        def _(w):
            for j in range(8):
                pltpu.async_copy(ones_v.at[0], acc.at[idx_v.at[w + j]],
                                 sem, add=True)
            for j in range(8):
                pltpu.make_async_copy(ones_v.at[0], acc.at[idx_v.at[w + j]],
                                      sem).wait()

        plsc.subcore_barrier()
        pltpu.sync_copy(acc.at[pl.ds(s * RPT, RPT)],
                        out_hbm.at[c, pl.ds(s * RPT, RPT)])

    return k(e4)


def _sc_propagate(g_pad, e4, d_ch, d_wb=None):
    d_wb = d_ch if d_wb is None else d_wb

    @functools.partial(
        pl.kernel,
        out_type=jax.ShapeDtypeStruct((NC, N_PAD, d_wb), jnp.float32),
        mesh=_MESH,
        scratch_types=[
            pltpu.VMEM_SHARED((N_PAD, d_ch), jnp.float32),
            pltpu.VMEM((NWIN, WIN), jnp.int32),
            pltpu.VMEM((1, WIN), jnp.int32),
            pltpu.VMEM((1, WIN), jnp.int32),
            pltpu.VMEM((WIN, d_ch), jnp.float32),
            pltpu.VMEM((WIN, d_ch), jnp.float32),
            pltpu.VMEM((32, d_ch), jnp.float32),
            pltpu.SemaphoreType.DMA,
            pltpu.SemaphoreType.DMA,
            pltpu.SemaphoreType.DMA,
            pltpu.SemaphoreType.DMA,
            pltpu.SemaphoreType.DMA,
        ],
    )
    def k(g_hbm, e_hbm, out_hbm, acc, sidx, didx_a, didx_b,
          rows_a, rows_b, z_v, sem_i, sem_a, sem_b, sem_da, sem_db):
        c = lax.axis_index("c")
        s = lax.axis_index("s")
        wid = c * NS + s
        pltpu.async_copy(e_hbm.at[0, wid], sidx, sem_i)

        @pl.loop(0, 32)
        def _(r):
            @pl.loop(0, d_ch, step=16)
            def _(k16):
                z_v[r, pl.ds(k16, 16)] = jnp.zeros((16,), jnp.float32)

        pltpu.make_async_copy(e_hbm.at[0, wid], sidx, sem_i).wait()
        pltpu.async_copy(e_hbm.at[1, wid, pl.ds(0, 1)], didx_a, sem_da)
        pltpu.async_copy(e_hbm.at[1, wid, pl.ds(1, 1)], didx_b, sem_db)
        pltpu.async_copy(g_hbm.at[sidx.at[0]], rows_a, sem_a)
        pltpu.async_copy(g_hbm.at[sidx.at[1]], rows_b, sem_b)

        @pl.loop(0, RPT, step=32)
        def _(r0):
            pltpu.sync_copy(z_v, acc.at[pl.ds(s * RPT + r0, 32)])

        plsc.subcore_barrier()

        @pl.loop(0, NWIN, step=2)
        def _(w):
            pltpu.make_async_copy(g_hbm.at[sidx.at[w]], rows_a, sem_a).wait()
            pltpu.make_async_copy(e_hbm.at[1, wid, pl.ds(w, 1)], didx_a,
                                  sem_da).wait()
            pltpu.sync_copy(rows_a, acc.at[didx_a.at[0]], add=True)

            @pl.when(w + 2 < NWIN)
            def _():
                pltpu.async_copy(e_hbm.at[1, wid, pl.ds(w + 2, 1)], didx_a,
                                 sem_da)
                pltpu.async_copy(g_hbm.at[sidx.at[w + 2]], rows_a, sem_a)

            pltpu.make_async_copy(g_hbm.at[sidx.at[w + 1]], rows_b,
                                  sem_b).wait()
            pltpu.make_async_copy(e_hbm.at[1, wid, pl.ds(w + 1, 1)], didx_b,
                                  sem_db).wait()
            pltpu.sync_copy(rows_b, acc.at[didx_b.at[0]], add=True)

            @pl.when(w + 3 < NWIN)
            def _():
                pltpu.async_copy(e_hbm.at[1, wid, pl.ds(w + 3, 1)], didx_b,
                                 sem_db)
                pltpu.async_copy(g_hbm.at[sidx.at[w + 3]], rows_b, sem_b)

        plsc.subcore_barrier()
        if d_wb == d_ch:
            pltpu.sync_copy(acc.at[pl.ds(s * RPT, RPT)],
                            out_hbm.at[c, pl.ds(s * RPT, RPT)])
        else:
            pltpu.sync_copy(acc.at[pl.ds(s * RPT, RPT), pl.ds(0, d_wb)],
                            out_hbm.at[c, pl.ds(s * RPT, RPT)])

    return k(g_pad, e4)


def _tc_mm1(x, W1):
    blk = 2000

    def body(x_ref, w_ref, h_ref):
        h_ref[...] = _dot_b3(x_ref[...], w_ref[...])

    return pl.pallas_call(
        body,
        grid=(N_NODES // blk,),
        in_specs=[pl.BlockSpec((blk, IN_CH), lambda r: (r, 0)),
                  pl.BlockSpec((IN_CH, IN_CH), lambda r: (0, 0))],
        out_specs=pl.BlockSpec((blk, IN_CH), lambda r: (r, 0)),
        out_shape=jax.ShapeDtypeStruct((N_PAD, IN_CH), jnp.float32),
    )(x, W1)


def _tc_scale(h1, cnt):
    blk = 1280

    def body(h_ref, cnt_ref, g_ref, dinv_ref):
        c = cnt_ref[...]
        deg_row = c[0:1, :] + c[1:2, :] + 1.0
        dinv_col = jnp.reshape(lax.rsqrt(deg_row), (blk, 1))
        dinv_ref[...] = dinv_col
        g_ref[...] = h_ref[...] * dinv_col

    return pl.pallas_call(
        body,
        grid=(N_PAD // blk,),
        in_specs=[pl.BlockSpec((blk, IN_CH), lambda r: (r, 0)),
                  pl.BlockSpec((NC, blk), lambda r: (0, r))],
        out_specs=(pl.BlockSpec((blk, IN_CH), lambda r: (r, 0)),
                   pl.BlockSpec((blk, 1), lambda r: (r, 0))),
        out_shape=(jax.ShapeDtypeStruct((N_PAD, IN_CH), jnp.float32),
                   jax.ShapeDtypeStruct((N_PAD, 1), jnp.float32)),
    )(h1, cnt)


def _tc_layer(p, g, dinv, b, W):
    d_out = W.shape[1]
    d_in = g.shape[1]
    blk = 1280

    def body(p_ref, g_ref, dinv_ref, b_ref, w_ref, o_ref):
        dinv = dinv_ref[...]
        xin = jax.nn.relu(dinv * (p_ref[0] + p_ref[1] + g_ref[...])
                          + b_ref[...])
        h = _dot_b3(xin, w_ref[...])
        if d_out == d_in:
            o_ref[...] = h * dinv
        else:
            o_ref[:, 0:d_out] = h * dinv

    return pl.pallas_call(
        body,
        grid=(N_PAD // blk,),
        in_specs=[pl.BlockSpec((NC, blk, d_in), lambda r: (0, r, 0)),
                  pl.BlockSpec((blk, d_in), lambda r: (r, 0)),
                  pl.BlockSpec((blk, 1), lambda r: (r, 0)),
                  pl.BlockSpec((1, d_in), lambda r: (0, 0)),
                  pl.BlockSpec((d_in, d_out), lambda r: (0, 0))],
        out_specs=pl.BlockSpec((blk, d_in), lambda r: (r, 0)),
        out_shape=jax.ShapeDtypeStruct((N_PAD, d_in), jnp.float32),
    )(p, g, dinv, b, W)


def _tc_final(p, g, dinv, b, d_out):
    d_rd = p.shape[2]
    blk = 2000

    def body(p_ref, g_ref, dinv_ref, b_ref, o_ref):
        v = dinv_ref[...] * (p_ref[0] + p_ref[1] + g_ref[:, 0:d_rd])
        o_ref[...] = v[:, 0:d_out] + b_ref[...]

    return pl.pallas_call(
        body,
        grid=(N_NODES // blk,),
        in_specs=[pl.BlockSpec((NC, blk, d_rd), lambda r: (0, r, 0)),
                  pl.BlockSpec((blk, g.shape[1]), lambda r: (r, 0)),
                  pl.BlockSpec((blk, 1), lambda r: (r, 0)),
                  pl.BlockSpec((1, d_out), lambda r: (0, 0))],
        out_specs=pl.BlockSpec((blk, d_out), lambda r: (r, 0)),
        out_shape=jax.ShapeDtypeStruct((N_NODES, d_out), jnp.float32),
    )(p, g, dinv, b)


def kernel(x, edge_index, W1, b1, W2, b2, W3, b3):
    e2 = edge_index.astype(jnp.int32).reshape(2, NW, N_EDGES // NW)
    n_fill = EPW - N_EDGES // NW
    fill = jnp.broadcast_to(
        (N_NODES + jnp.arange(n_fill, dtype=jnp.int32) % (N_PAD - N_NODES)
         )[None, None, :], (2, NW, n_fill))
    e4 = jnp.concatenate([e2, fill], axis=2).reshape(2, NW, NWIN, WIN)

    cnt = _sc_degree(e4)
    h1 = _tc_mm1(x, W1)
    g1, dinv = _tc_scale(h1, cnt)
    p1 = _sc_propagate(g1, e4, 128)
    g2 = _tc_layer(p1, g1, dinv, b1[None, :], W2)
    p2 = _sc_propagate(g2, e4, 128)
    g3 = _tc_layer(p2, g2, dinv, b2[None, :], W3)
    p3 = _sc_propagate(g3, e4, 128)
    return _tc_final(p3, g3, dinv, b3[None, :], W3.shape[1])


# speedup vs baseline: 1.0140x; 1.0140x over previous
import functools

import jax
import jax.numpy as jnp
from jax import lax
from jax.experimental import pallas as pl
from jax.experimental.pallas import tpu as pltpu
from jax.experimental.pallas import tpu_sc as plsc

N_NODES = 10000
N_PAD = 10240
N_EDGES = 320000
IN_CH = 128
NC = 2
NS = 16
NW = NC * NS
WIN = 128
NWIN = 80
EPW = NWIN * WIN
RPT = N_PAD // NS

_MESH = plsc.VectorSubcoreMesh(core_axis_name="c", subcore_axis_name="s")


def _dot_b3(a, b):
    ah = a.astype(jnp.bfloat16)
    al = (a - ah.astype(jnp.float32)).astype(jnp.bfloat16)
    bh = b.astype(jnp.bfloat16)
    bl = (b - bh.astype(jnp.float32)).astype(jnp.bfloat16)
    d = lambda u, v: jnp.dot(u, v, preferred_element_type=jnp.float32)
    return d(ah, bh) + d(ah, bl) + d(al, bh)


def _sc_degree(e4):
    @functools.partial(
        pl.kernel,
        out_type=jax.ShapeDtypeStruct((NC, N_PAD), jnp.float32),
        mesh=_MESH,
        scratch_types=[
            pltpu.VMEM_SHARED((N_PAD,), jnp.float32),
            pltpu.VMEM((NWIN, WIN), jnp.int32),
            pltpu.VMEM((1, WIN), jnp.float32),
            pltpu.VMEM((RPT,), jnp.float32),
            pltpu.SemaphoreType.DMA,
        ],
    )
    def k(e_hbm, out_hbm, acc, idx_v, ones_v, z_v, sem):
        c = lax.axis_index("c")
        s = lax.axis_index("s")
        wid = c * NS + s
        pltpu.async_copy(e_hbm.at[1, wid], idx_v, sem)

        @pl.loop(0, WIN, step=16)
        def _(i):
            ones_v[0, pl.ds(i, 16)] = jnp.ones((16,), jnp.float32)

        @pl.loop(0, RPT, step=16)
        def _(i):
            z_v[pl.ds(i, 16)] = jnp.zeros((16,), jnp.float32)

        pltpu.sync_copy(z_v, acc.at[pl.ds(s * RPT, RPT)])
        plsc.subcore_barrier()
        pltpu.make_async_copy(e_hbm.at[1, wid], idx_v, sem).wait()

        @pl.loop(0, NWIN, step=8)
        def _(w):
            for j in range(8):
                pltpu.async_copy(ones_v.at[0], acc.at[idx_v.at[w + j]],
                                 sem, add=True)
            for j in range(8):
                pltpu.make_async_copy(ones_v.at[0], acc.at[idx_v.at[w + j]],
                                      sem).wait()

        plsc.subcore_barrier()
        pltpu.sync_copy(acc.at[pl.ds(s * RPT, RPT)],
                        out_hbm.at[c, pl.ds(s * RPT, RPT)])

    return k(e4)


def _sc_propagate(g_pad, e4, d_ch, d_wb=None):
    d_wb = d_ch if d_wb is None else d_wb

    @functools.partial(
        pl.kernel,
        out_type=jax.ShapeDtypeStruct((NC, N_PAD, d_wb), jnp.float32),
        mesh=_MESH,
        scratch_types=[
            pltpu.VMEM_SHARED((N_PAD, d_ch), jnp.float32),
            pltpu.VMEM((NWIN, WIN), jnp.int32),
            pltpu.VMEM((1, WIN), jnp.int32),
            pltpu.VMEM((1, WIN), jnp.int32),
            pltpu.VMEM((WIN, d_ch), jnp.float32),
            pltpu.VMEM((WIN, d_ch), jnp.float32),
            pltpu.VMEM((32, d_ch), jnp.float32),
            pltpu.SemaphoreType.DMA,
            pltpu.SemaphoreType.DMA,
            pltpu.SemaphoreType.DMA,
            pltpu.SemaphoreType.DMA,
            pltpu.SemaphoreType.DMA,
        ],
    )
    def k(g_hbm, e_hbm, out_hbm, acc, sidx, didx_a, didx_b,
          rows_a, rows_b, z_v, sem_i, sem_a, sem_b, sem_da, sem_db):
        c = lax.axis_index("c")
        s = lax.axis_index("s")
        wid = c * NS + s
        pltpu.async_copy(e_hbm.at[0, wid], sidx, sem_i)

        @pl.loop(0, 32)
        def _(r):
            @pl.loop(0, d_ch, step=16)
            def _(k16):
                z_v[r, pl.ds(k16, 16)] = jnp.zeros((16,), jnp.float32)

        pltpu.make_async_copy(e_hbm.at[0, wid], sidx, sem_i).wait()
        pltpu.async_copy(e_hbm.at[1, wid, pl.ds(0, 1)], didx_a, sem_da)
        pltpu.async_copy(e_hbm.at[1, wid, pl.ds(1, 1)], didx_b, sem_db)
        pltpu.async_copy(g_hbm.at[sidx.at[0]], rows_a, sem_a)
        pltpu.async_copy(g_hbm.at[sidx.at[1]], rows_b, sem_b)

        @pl.loop(0, RPT, step=32)
        def _(r0):
            pltpu.sync_copy(z_v, acc.at[pl.ds(s * RPT + r0, 32)])

        plsc.subcore_barrier()

        @pl.loop(0, NWIN, step=2)
        def _(w):
            pltpu.make_async_copy(g_hbm.at[sidx.at[w]], rows_a, sem_a).wait()
            pltpu.make_async_copy(e_hbm.at[1, wid, pl.ds(w, 1)], didx_a,
                                  sem_da).wait()
            pltpu.sync_copy(rows_a, acc.at[didx_a.at[0]], add=True)

            @pl.when(w + 2 < NWIN)
            def _():
                pltpu.async_copy(e_hbm.at[1, wid, pl.ds(w + 2, 1)], didx_a,
                                 sem_da)
                pltpu.async_copy(g_hbm.at[sidx.at[w + 2]], rows_a, sem_a)

            pltpu.make_async_copy(g_hbm.at[sidx.at[w + 1]], rows_b,
                                  sem_b).wait()
            pltpu.make_async_copy(e_hbm.at[1, wid, pl.ds(w + 1, 1)], didx_b,
                                  sem_db).wait()
            pltpu.sync_copy(rows_b, acc.at[didx_b.at[0]], add=True)

            @pl.when(w + 3 < NWIN)
            def _():
                pltpu.async_copy(e_hbm.at[1, wid, pl.ds(w + 3, 1)], didx_b,
                                 sem_db)
                pltpu.async_copy(g_hbm.at[sidx.at[w + 3]], rows_b, sem_b)

        plsc.subcore_barrier()
        if d_wb == d_ch:
            pltpu.sync_copy(acc.at[pl.ds(s * RPT, RPT)],
                            out_hbm.at[c, pl.ds(s * RPT, RPT)])
        else:
            pltpu.sync_copy(acc.at[pl.ds(s * RPT, RPT), pl.ds(0, d_wb)],
                            out_hbm.at[c, pl.ds(s * RPT, RPT)])

    return k(g_pad, e4)


def _tc_mm1(x, W1):
    blk = 2000

    def body(x_ref, w_ref, h_ref):
        h_ref[...] = _dot_b3(x_ref[...], w_ref[...])

    return pl.pallas_call(
        body,
        grid=(N_NODES // blk,),
        in_specs=[pl.BlockSpec((blk, IN_CH), lambda r: (r, 0)),
                  pl.BlockSpec((IN_CH, IN_CH), lambda r: (0, 0))],
        out_specs=pl.BlockSpec((blk, IN_CH), lambda r: (r, 0)),
        out_shape=jax.ShapeDtypeStruct((N_PAD, IN_CH), jnp.float32),
    )(x, W1)


def _tc_scale(h1, cnt):
    blk = 2560

    def body(h_ref, cnt_ref, g_ref, dinv_ref):
        c = cnt_ref[...]
        deg_row = c[0:1, :] + c[1:2, :] + 1.0
        dinv_col = jnp.reshape(lax.rsqrt(deg_row), (blk, 1))
        dinv_ref[...] = dinv_col
        g_ref[...] = h_ref[...] * dinv_col

    return pl.pallas_call(
        body,
        grid=(N_PAD // blk,),
        in_specs=[pl.BlockSpec((blk, IN_CH), lambda r: (r, 0)),
                  pl.BlockSpec((NC, blk), lambda r: (0, r))],
        out_specs=(pl.BlockSpec((blk, IN_CH), lambda r: (r, 0)),
                   pl.BlockSpec((blk, 1), lambda r: (r, 0))),
        out_shape=(jax.ShapeDtypeStruct((N_PAD, IN_CH), jnp.float32),
                   jax.ShapeDtypeStruct((N_PAD, 1), jnp.float32)),
    )(h1, cnt)


def _tc_layer(p, g, dinv, b, W):
    d_out = W.shape[1]
    d_in = g.shape[1]
    blk = 2560

    def body(p_ref, g_ref, dinv_ref, b_ref, w_ref, o_ref):
        dinv = dinv_ref[...]
        xin = jax.nn.relu(dinv * (p_ref[0] + p_ref[1] + g_ref[...])
                          + b_ref[...])
        h = _dot_b3(xin, w_ref[...])
        if d_out == d_in:
            o_ref[...] = h * dinv
        else:
            o_ref[:, 0:d_out] = h * dinv

    return pl.pallas_call(
        body,
        grid=(N_PAD // blk,),
        in_specs=[pl.BlockSpec((NC, blk, d_in), lambda r: (0, r, 0)),
                  pl.BlockSpec((blk, d_in), lambda r: (r, 0)),
                  pl.BlockSpec((blk, 1), lambda r: (r, 0)),
                  pl.BlockSpec((1, d_in), lambda r: (0, 0)),
                  pl.BlockSpec((d_in, d_out), lambda r: (0, 0))],
        out_specs=pl.BlockSpec((blk, d_in), lambda r: (r, 0)),
        out_shape=jax.ShapeDtypeStruct((N_PAD, d_in), jnp.float32),
    )(p, g, dinv, b, W)


def _tc_final(p, g, dinv, b, d_out):
    d_rd = p.shape[2]
    blk = 2000

    def body(p_ref, g_ref, dinv_ref, b_ref, o_ref):
        v = dinv_ref[...] * (p_ref[0] + p_ref[1] + g_ref[:, 0:d_rd])
        o_ref[...] = v[:, 0:d_out] + b_ref[...]

    return pl.pallas_call(
        body,
        grid=(N_NODES // blk,),
        in_specs=[pl.BlockSpec((NC, blk, d_rd), lambda r: (0, r, 0)),
                  pl.BlockSpec((blk, g.shape[1]), lambda r: (r, 0)),
                  pl.BlockSpec((blk, 1), lambda r: (r, 0)),
                  pl.BlockSpec((1, d_out), lambda r: (0, 0))],
        out_specs=pl.BlockSpec((blk, d_out), lambda r: (r, 0)),
        out_shape=jax.ShapeDtypeStruct((N_NODES, d_out), jnp.float32),
    )(p, g, dinv, b)


def kernel(x, edge_index, W1, b1, W2, b2, W3, b3):
    e2 = edge_index.astype(jnp.int32).reshape(2, NW, N_EDGES // NW)
    n_fill = EPW - N_EDGES // NW
    fill = jnp.broadcast_to(
        (N_NODES + jnp.arange(n_fill, dtype=jnp.int32) % (N_PAD - N_NODES)
         )[None, None, :], (2, NW, n_fill))
    e4 = jnp.concatenate([e2, fill], axis=2).reshape(2, NW, NWIN, WIN)

    cnt = _sc_degree(e4)
    h1 = _tc_mm1(x, W1)
    g1, dinv = _tc_scale(h1, cnt)
    p1 = _sc_propagate(g1, e4, 128)
    g2 = _tc_layer(p1, g1, dinv, b1[None, :], W2)
    p2 = _sc_propagate(g2, e4, 128)
    g3 = _tc_layer(p2, g2, dinv, b2[None, :], W3)
    p3 = _sc_propagate(g3, e4, 128)
    return _tc_final(p3, g3, dinv, b3[None, :], W3.shape[1])
